# Initial kernel scaffold; baseline (speedup 1.0000x reference)
import jax
import jax.numpy as jnp
from jax.experimental import pallas as pl


def kernel(nodes, senders, receivers, n_node, n_edge, W1, b1, W2, b2):
    raise NotImplementedError("write your pallas kernel here")


# speedup vs baseline: 220.1206x; 220.1206x over previous
import functools

import jax
import jax.numpy as jnp
from jax import lax
from jax.experimental import pallas as pl
from jax.experimental.pallas import tpu as pltpu
from jax.experimental.pallas import tpu_sc as plsc

N = 10000
G = 8
NPG = N // G
E = 320000
D = 128
C = 64
K = 16
HID = 32

NC = 2
NS = 16
NW = NC * NS
EPW = E // NW
CH = 80
NCH = EPW // CH
SRW = 624
TAIL = N - NS * SRW


def _mlp_body(x_ref, w1_ref, b1_ref, w2_ref, b2_ref, assign_ref, coarse_ref):
    x = x_ref[0]
    h = jnp.dot(x, w1_ref[...], preferred_element_type=jnp.float32)
    h = jnp.maximum(h + b1_ref[0], 0.0)
    logits = jnp.dot(h, w2_ref[...], preferred_element_type=jnp.float32)
    logits = logits + b2_ref[0]
    m = jnp.max(logits, axis=-1, keepdims=True)
    e = jnp.exp(logits - m)
    a = e / jnp.sum(e, axis=-1, keepdims=True)
    assign_ref[0] = a
    coarse_ref[0] = lax.dot_general(
        a, x, (((0,), (0,)), ((), ())), preferred_element_type=jnp.float32)


def _stage1(nodes3, w1, b1, w2, b2):
    return pl.pallas_call(
        _mlp_body,
        grid=(G,),
        in_specs=[
            pl.BlockSpec((1, NPG, D), lambda i: (i, 0, 0)),
            pl.BlockSpec((D, HID), lambda i: (0, 0)),
            pl.BlockSpec((1, HID), lambda i: (0, 0)),
            pl.BlockSpec((HID, C), lambda i: (0, 0)),
            pl.BlockSpec((1, C), lambda i: (0, 0)),
        ],
        out_specs=[
            pl.BlockSpec((1, NPG, C), lambda i: (i, 0, 0)),
            pl.BlockSpec((1, C, D), lambda i: (i, 0, 0)),
        ],
        out_shape=[
            jax.ShapeDtypeStruct((G, NPG, C), jnp.float32),
            jax.ShapeDtypeStruct((G, C, D), jnp.float32),
        ],
    )(nodes3, w1, b1, w2, b2)


def _edge_body(assign_hbm, senders_hbm, receivers_hbm, zeros_hbm, t_hbm,
               sidx_v, ridx_v, rows_v, t_sh, sem):
    cid = lax.axis_index("c")
    sid = lax.axis_index("s")
    wid = sid * NC + cid
    stripe = pl.multiple_of(sid * SRW, 8)
    pltpu.sync_copy(zeros_hbm, t_sh.at[pl.ds(stripe, SRW)])

    @pl.when(sid == NS - 1)
    def _zero_tail():
        pltpu.sync_copy(zeros_hbm.at[pl.ds(0, TAIL)],
                        t_sh.at[pl.ds(NS * SRW, TAIL)])

    plsc.subcore_barrier()

    def chunk(c, carry):
        off = wid * EPW + c * CH
        pltpu.sync_copy(senders_hbm.at[pl.ds(off, CH)], sidx_v)
        pltpu.sync_copy(receivers_hbm.at[pl.ds(off, CH)], ridx_v)
        pltpu.async_copy(assign_hbm.at[ridx_v], rows_v, sem).wait()
        pltpu.sync_copy(rows_v, t_sh.at[sidx_v], add=True)
        return carry

    lax.fori_loop(0, NCH, chunk, 0)
    plsc.subcore_barrier()
    pltpu.sync_copy(t_sh.at[pl.ds(stripe, SRW)],
                    t_hbm.at[cid, pl.ds(stripe, SRW)])

    @pl.when(sid == NS - 1)
    def _write_tail():
        pltpu.sync_copy(t_sh.at[pl.ds(NS * SRW, TAIL)],
                        t_hbm.at[cid, pl.ds(NS * SRW, TAIL)])


def _stage2(assignments, senders, receivers, zeros):
    run = pl.kernel(
        _edge_body,
        out_type=jax.ShapeDtypeStruct((NC, N, C), jnp.float32),
        mesh=plsc.VectorSubcoreMesh(core_axis_name="c", subcore_axis_name="s"),
        scratch_types=[
            pltpu.VMEM((CH,), jnp.int32),
            pltpu.VMEM((CH,), jnp.int32),
            pltpu.VMEM((CH, C), jnp.float32),
            pltpu.VMEM_SHARED((N, C), jnp.float32),
            pltpu.SemaphoreType.DMA,
        ],
        compiler_params=pltpu.CompilerParams(use_tc_tiling_on_sc=False),
    )
    return run(assignments, senders, receivers, zeros)


def _adj_body(a_ref, t_ref, vals_ref, idx_ref):
    a = a_ref[0]
    t = t_ref[0, 0] + t_ref[1, 0]
    adj = lax.dot_general(
        a, t, (((0,), (0,)), ((), ())), preferred_element_type=jnp.float32)
    va = adj[:, :, None]
    vb = adj[:, None, :]
    jj = lax.broadcasted_iota(jnp.int32, (C, C, C), 1)
    jp = lax.broadcasted_iota(jnp.int32, (C, C, C), 2)
    cmp = (vb > va) | ((vb == va) & (jp < jj))
    rank = jnp.sum(cmp.astype(jnp.int32), axis=2)
    rr = lax.broadcasted_iota(jnp.int32, (C, C, C), 2)
    onehot = (rank[:, :, None] == rr)
    vals_ref[0] = jnp.sum(jnp.where(onehot, adj[:, :, None], 0.0), axis=1)
    idx_ref[0] = jnp.sum(jnp.where(onehot, jj, 0), axis=1).astype(jnp.int32)


def _stage3(assign3, t4):
    return pl.pallas_call(
        _adj_body,
        grid=(G,),
        in_specs=[
            pl.BlockSpec((1, NPG, C), lambda i: (i, 0, 0)),
            pl.BlockSpec((NC, 1, NPG, C), lambda i: (0, i, 0, 0)),
        ],
        out_specs=[
            pl.BlockSpec((1, C, C), lambda i: (i, 0, 0)),
            pl.BlockSpec((1, C, C), lambda i: (i, 0, 0)),
        ],
        out_shape=[
            jax.ShapeDtypeStruct((G, C, C), jnp.float32),
            jax.ShapeDtypeStruct((G, C, C), jnp.int32),
        ],
    )(assign3, t4)


def kernel(nodes, senders, receivers, n_node, n_edge, W1, b1, W2, b2):
    del n_node, n_edge
    nodes3 = nodes.reshape(G, NPG, D)
    assign3, coarse3 = _stage1(nodes3, W1, b1.reshape(1, HID), W2,
                               b2.reshape(1, C))
    assignments = assign3.reshape(N, C)
    zeros = jnp.zeros((SRW, C), jnp.float32)
    t = _stage2(assignments, senders, receivers, zeros)
    vals_s, idx_s = _stage3(assign3, t.reshape(NC, G, NPG, C))
    top_vals = vals_s[:, :, :K]
    top_idx = idx_s[:, :, :K]
    batch_offset = jnp.arange(G, dtype=jnp.int32)[:, None] * C
    c_senders = (jnp.repeat(jnp.arange(C, dtype=jnp.int32), K)[None, :]
                 + batch_offset).reshape(-1)
    c_receivers = (top_idx.reshape(G, C * K) + batch_offset).reshape(-1)
    c_edge_weights = top_vals.reshape(-1, 1)
    coarse_nodes = coarse3.reshape(G * C, D)
    return (coarse_nodes, c_senders, c_receivers, c_edge_weights,
            assignments.astype(jnp.float32))


# speedup vs baseline: 552.2684x; 2.5089x over previous
import functools

import jax
import jax.numpy as jnp
from jax import lax
from jax.experimental import pallas as pl
from jax.experimental.pallas import tpu as pltpu
from jax.experimental.pallas import tpu_sc as plsc

N = 10000
G = 8
NPG = N // G
E = 320000
D = 128
C = 64
K = 16
HID = 32

NC = 2
NS = 16
NW = NC * NS
EPW = E // NW
CH = 80
NCH = EPW // CH
NBUF = 5
SRW = 624
TAIL = N - NS * SRW


def _mlp_body(x_ref, w1_ref, b1_ref, w2_ref, b2_ref, assign_ref, coarse_ref):
    x = x_ref[0]
    h = jnp.dot(x, w1_ref[...], preferred_element_type=jnp.float32)
    h = jnp.maximum(h + b1_ref[0], 0.0)
    logits = jnp.dot(h, w2_ref[...], preferred_element_type=jnp.float32)
    logits = logits + b2_ref[0]
    m = jnp.max(logits, axis=-1, keepdims=True)
    e = jnp.exp(logits - m)
    a = e / jnp.sum(e, axis=-1, keepdims=True)
    assign_ref[0] = a
    coarse_ref[0] = lax.dot_general(
        a, x, (((0,), (0,)), ((), ())), preferred_element_type=jnp.float32)


def _stage1(nodes3, w1, b1, w2, b2):
    return pl.pallas_call(
        _mlp_body,
        grid=(G,),
        in_specs=[
            pl.BlockSpec((1, NPG, D), lambda i: (i, 0, 0)),
            pl.BlockSpec((D, HID), lambda i: (0, 0)),
            pl.BlockSpec((1, HID), lambda i: (0, 0)),
            pl.BlockSpec((HID, C), lambda i: (0, 0)),
            pl.BlockSpec((1, C), lambda i: (0, 0)),
        ],
        out_specs=[
            pl.BlockSpec((1, NPG, C), lambda i: (i, 0, 0)),
            pl.BlockSpec((1, C, D), lambda i: (i, 0, 0)),
        ],
        out_shape=[
            jax.ShapeDtypeStruct((G, NPG, C), jnp.float32),
            jax.ShapeDtypeStruct((G, C, D), jnp.float32),
        ],
    )(nodes3, w1, b1, w2, b2)


def _edge_body(assign_hbm, senders_hbm, receivers_hbm, zeros_hbm, t_hbm,
               sidx_v, ridx_v, rows_v, t_sh, gsem):
    cid = lax.axis_index("c")
    sid = lax.axis_index("s")
    wid = sid * NC + cid
    stripe = pl.multiple_of(sid * SRW, 8)
    pltpu.sync_copy(zeros_hbm, t_sh.at[pl.ds(stripe, SRW)])

    @pl.when(sid == NS - 1)
    def _zero_tail():
        pltpu.sync_copy(zeros_hbm.at[pl.ds(0, TAIL)],
                        t_sh.at[pl.ds(NS * SRW, TAIL)])

    pltpu.sync_copy(senders_hbm.at[wid], sidx_v)
    pltpu.sync_copy(receivers_hbm.at[wid], ridx_v)
    plsc.subcore_barrier()

    for b in range(NBUF):
        pltpu.make_async_copy(assign_hbm.at[ridx_v.at[b]], rows_v.at[b],
                              gsem.at[b]).start()

    def outer(g, carry):
        for b in range(NBUF):
            c = g * NBUF + b
            pltpu.make_async_copy(assign_hbm.at[ridx_v.at[b]], rows_v.at[b],
                                  gsem.at[b]).wait()
            pltpu.sync_copy(rows_v.at[b], t_sh.at[sidx_v.at[c]], add=True)
            cn = c + NBUF

            @pl.when(cn < NCH)
            def _prefetch():
                pltpu.make_async_copy(assign_hbm.at[ridx_v.at[cn]],
                                      rows_v.at[b], gsem.at[b]).start()
        return carry

    lax.fori_loop(0, NCH // NBUF, outer, 0)
    plsc.subcore_barrier()
    pltpu.sync_copy(t_sh.at[pl.ds(stripe, SRW)],
                    t_hbm.at[cid, pl.ds(stripe, SRW)])

    @pl.when(sid == NS - 1)
    def _write_tail():
        pltpu.sync_copy(t_sh.at[pl.ds(NS * SRW, TAIL)],
                        t_hbm.at[cid, pl.ds(NS * SRW, TAIL)])


def _stage2(assignments, senders, receivers, zeros):
    run = pl.kernel(
        _edge_body,
        out_type=jax.ShapeDtypeStruct((NC, N, C), jnp.float32),
        mesh=plsc.VectorSubcoreMesh(core_axis_name="c", subcore_axis_name="s"),
        scratch_types=[
            pltpu.VMEM((NCH, CH), jnp.int32),
            pltpu.VMEM((NCH, CH), jnp.int32),
            pltpu.VMEM((NBUF, CH, C), jnp.float32),
            pltpu.VMEM_SHARED((N, C), jnp.float32),
            pltpu.SemaphoreType.DMA((NBUF,)),
        ],
        compiler_params=pltpu.CompilerParams(use_tc_tiling_on_sc=False),
    )
    return run(assignments, senders, receivers, zeros)


def _adj_body(a_ref, t_ref, vals_ref, idx_ref):
    a = a_ref[0]
    t = t_ref[0, 0] + t_ref[1, 0]
    adj = lax.dot_general(
        a, t, (((0,), (0,)), ((), ())), preferred_element_type=jnp.float32)
    va = adj[:, :, None]
    vb = adj[:, None, :]
    jj = lax.broadcasted_iota(jnp.int32, (C, C, C), 1)
    jp = lax.broadcasted_iota(jnp.int32, (C, C, C), 2)
    cmp = (vb > va) | ((vb == va) & (jp < jj))
    rank = jnp.sum(cmp.astype(jnp.int32), axis=2)
    rr = lax.broadcasted_iota(jnp.int32, (C, C, C), 2)
    onehot = (rank[:, :, None] == rr)
    vals_ref[0] = jnp.sum(jnp.where(onehot, adj[:, :, None], 0.0), axis=1)
    idx_ref[0] = jnp.sum(jnp.where(onehot, jj, 0), axis=1).astype(jnp.int32)


def _stage3(assign3, t4):
    return pl.pallas_call(
        _adj_body,
        grid=(G,),
        in_specs=[
            pl.BlockSpec((1, NPG, C), lambda i: (i, 0, 0)),
            pl.BlockSpec((NC, 1, NPG, C), lambda i: (0, i, 0, 0)),
        ],
        out_specs=[
            pl.BlockSpec((1, C, C), lambda i: (i, 0, 0)),
            pl.BlockSpec((1, C, C), lambda i: (i, 0, 0)),
        ],
        out_shape=[
            jax.ShapeDtypeStruct((G, C, C), jnp.float32),
            jax.ShapeDtypeStruct((G, C, C), jnp.int32),
        ],
    )(assign3, t4)


def kernel(nodes, senders, receivers, n_node, n_edge, W1, b1, W2, b2):
    del n_node, n_edge
    nodes3 = nodes.reshape(G, NPG, D)
    assign3, coarse3 = _stage1(nodes3, W1, b1.reshape(1, HID), W2,
                               b2.reshape(1, C))
    assignments = assign3.reshape(N, C)
    zeros = jnp.zeros((SRW, C), jnp.float32)
    t = _stage2(assignments, senders.reshape(NW, NCH, CH),
                receivers.reshape(NW, NCH, CH), zeros)
    vals_s, idx_s = _stage3(assign3, t.reshape(NC, G, NPG, C))
    top_vals = vals_s[:, :, :K]
    top_idx = idx_s[:, :, :K]
    batch_offset = jnp.arange(G, dtype=jnp.int32)[:, None] * C
    c_senders = (jnp.repeat(jnp.arange(C, dtype=jnp.int32), K)[None, :]
                 + batch_offset).reshape(-1)
    c_receivers = (top_idx.reshape(G, C * K) + batch_offset).reshape(-1)
    c_edge_weights = top_vals.reshape(-1, 1)
    coarse_nodes = coarse3.reshape(G * C, D)
    return (coarse_nodes, c_senders, c_receivers, c_edge_weights,
            assignments.astype(jnp.float32))


# speedup vs baseline: 572.5613x; 1.0367x over previous
import functools

import jax
import jax.numpy as jnp
from jax import lax
from jax.experimental import pallas as pl
from jax.experimental.pallas import tpu as pltpu
from jax.experimental.pallas import tpu_sc as plsc

N = 10000
G = 8
NPG = N // G
E = 320000
D = 128
C = 64
K = 16
HID = 32

NC = 2
NS = 16
NW = NC * NS
EPW = E // NW
CH = 80
NCH = EPW // CH
NBUF = 5
SRW = 624
TAIL = N - NS * SRW


def _mlp_body(x_ref, w1_ref, b1_ref, w2_ref, b2_ref, assign_ref, coarse_ref):
    w1 = w1_ref[...]
    b1 = b1_ref[0]
    w2 = w2_ref[...]
    b2 = b2_ref[0]
    for g in range(G):
        x = x_ref[pl.ds(g * NPG, NPG), :]
        h = jnp.dot(x, w1, preferred_element_type=jnp.float32)
        h = jnp.maximum(h + b1, 0.0)
        logits = jnp.dot(h, w2, preferred_element_type=jnp.float32) + b2
        m = jnp.max(logits, axis=-1, keepdims=True)
        e = jnp.exp(logits - m)
        a = e / jnp.sum(e, axis=-1, keepdims=True)
        assign_ref[pl.ds(g * NPG, NPG), :] = a
        coarse_ref[pl.ds(g * C, C), :] = lax.dot_general(
            a, x, (((0,), (0,)), ((), ())), preferred_element_type=jnp.float32)


def _stage1(nodes, w1, b1, w2, b2):
    return pl.pallas_call(
        _mlp_body,
        out_shape=[
            jax.ShapeDtypeStruct((N, C), jnp.float32),
            jax.ShapeDtypeStruct((G * C, D), jnp.float32),
        ],
    )(nodes, w1, b1, w2, b2)


def _edge_body(assign_hbm, senders_hbm, receivers_hbm, zeros_hbm, t_hbm,
               sidx_v, ridx_v, sbuf_v, rows_v, t_sh, gsem):
    cid = lax.axis_index("c")
    sid = lax.axis_index("s")
    wid = sid * NC + cid
    stripe = pl.multiple_of(sid * SRW, 8)
    pltpu.sync_copy(zeros_hbm, t_sh.at[pl.ds(stripe, SRW)])

    @pl.when(sid == NS - 1)
    def _zero_tail():
        pltpu.sync_copy(zeros_hbm.at[pl.ds(0, TAIL)],
                        t_sh.at[pl.ds(NS * SRW, TAIL)])

    pltpu.sync_copy(senders_hbm.at[pl.ds(wid * EPW, EPW)], sidx_v)
    pltpu.sync_copy(receivers_hbm.at[pl.ds(wid * EPW, EPW)], ridx_v)
    plsc.subcore_barrier()

    for b in range(NBUF):
        pltpu.make_async_copy(assign_hbm.at[ridx_v.at[pl.ds(b * CH, CH)]],
                              rows_v.at[b], gsem.at[b]).start()

    def outer(g, carry):
        for b in range(NBUF):
            c = g * NBUF + b
            pltpu.make_async_copy(assign_hbm.at[ridx_v.at[pl.ds(b * CH, CH)]],
                                  rows_v.at[b], gsem.at[b]).wait()
            for j in range(CH // 16):
                sbuf_v[pl.ds(j * 16, 16)] = sidx_v[pl.ds(c * CH + j * 16, 16)]
            pltpu.sync_copy(rows_v.at[b], t_sh.at[sbuf_v], add=True)
            cn = c + NBUF

            @pl.when(cn < NCH)
            def _prefetch():
                pltpu.make_async_copy(
                    assign_hbm.at[ridx_v.at[pl.ds(cn * CH, CH)]],
                    rows_v.at[b], gsem.at[b]).start()
        return carry

    lax.fori_loop(0, NCH // NBUF, outer, 0)
    plsc.subcore_barrier()
    pltpu.sync_copy(t_sh.at[pl.ds(stripe, SRW)],
                    t_hbm.at[cid, pl.ds(stripe, SRW)])

    @pl.when(sid == NS - 1)
    def _write_tail():
        pltpu.sync_copy(t_sh.at[pl.ds(NS * SRW, TAIL)],
                        t_hbm.at[cid, pl.ds(NS * SRW, TAIL)])


def _stage2(assignments, senders, receivers, zeros):
    run = pl.kernel(
        _edge_body,
        out_type=jax.ShapeDtypeStruct((NC, N, C), jnp.float32),
        mesh=plsc.VectorSubcoreMesh(core_axis_name="c", subcore_axis_name="s"),
        scratch_types=[
            pltpu.VMEM((EPW,), jnp.int32),
            pltpu.VMEM((EPW,), jnp.int32),
            pltpu.VMEM((CH,), jnp.int32),
            pltpu.VMEM((NBUF, CH, C), jnp.float32),
            pltpu.VMEM_SHARED((N, C), jnp.float32),
            pltpu.SemaphoreType.DMA((NBUF,)),
        ],
        compiler_params=pltpu.CompilerParams(use_tc_tiling_on_sc=False),
    )
    return run(assignments, senders, receivers, zeros)


def _adj_body(a_ref, t_ref, vals_ref, idx_ref):
    t = t_ref[0] + t_ref[1]
    a = a_ref[...]
    for g in range(G):
        ag = a[g * NPG:(g + 1) * NPG]
        tg = t[g * NPG:(g + 1) * NPG]
        adj = lax.dot_general(
            ag, tg, (((0,), (0,)), ((), ())),
            preferred_element_type=jnp.float32)
        va = adj[:, :, None]
        vb = adj[:, None, :]
        jj = lax.broadcasted_iota(jnp.int32, (C, C, C), 1)
        jp = lax.broadcasted_iota(jnp.int32, (C, C, C), 2)
        cmp = (vb > va) | ((vb == va) & (jp < jj))
        rank = jnp.sum(cmp.astype(jnp.int32), axis=2)
        rr = lax.broadcasted_iota(jnp.int32, (C, C, C), 2)
        onehot = (rank[:, :, None] == rr)
        vals_ref[pl.ds(g * C, C), :] = jnp.sum(
            jnp.where(onehot, adj[:, :, None], 0.0), axis=1)
        idx_ref[pl.ds(g * C, C), :] = jnp.sum(
            jnp.where(onehot, jj, 0), axis=1).astype(jnp.int32)


def _stage3(assignments, t):
    return pl.pallas_call(
        _adj_body,
        out_shape=[
            jax.ShapeDtypeStruct((G * C, C), jnp.float32),
            jax.ShapeDtypeStruct((G * C, C), jnp.int32),
        ],
    )(assignments, t)


def kernel(nodes, senders, receivers, n_node, n_edge, W1, b1, W2, b2):
    del n_node, n_edge
    assignments, coarse_nodes = _stage1(nodes, W1, b1.reshape(1, HID), W2,
                                        b2.reshape(1, C))
    zeros = jnp.zeros((SRW, C), jnp.float32)
    t = _stage2(assignments, senders, receivers, zeros)
    vals_s, idx_s = _stage3(assignments, t)
    top_vals = vals_s[:, :K]
    top_idx = idx_s[:, :K]
    batch_offset = jnp.arange(G, dtype=jnp.int32)[:, None] * C
    c_senders = (jnp.repeat(jnp.arange(C, dtype=jnp.int32), K)[None, :]
                 + batch_offset).reshape(-1)
    c_receivers = (top_idx.reshape(G, C * K) + batch_offset).reshape(-1)
    c_edge_weights = top_vals.reshape(-1, 1)
    return (coarse_nodes, c_senders, c_receivers, c_edge_weights,
            assignments)


# speedup vs baseline: 626.3098x; 1.0939x over previous
import functools

import jax
import jax.numpy as jnp
from jax import lax
from jax.experimental import pallas as pl
from jax.experimental.pallas import tpu as pltpu
from jax.experimental.pallas import tpu_sc as plsc

N = 10000
G = 8
NPG = N // G
E = 320000
D = 128
C = 64
K = 16
HID = 32

NC = 2
NS = 16
NW = NC * NS
EPW = E // NW
CH = 80
NCH = EPW // CH
NBUF = 5
SRW = 624
TAIL = N - NS * SRW


def _mlp_body(x_ref, w1_ref, b1_ref, w2_ref, b2_ref, assign_ref, coarse_ref):
    w1 = w1_ref[...]
    b1 = b1_ref[0]
    w2 = w2_ref[...]
    b2 = b2_ref[0]
    for g in range(G):
        x = x_ref[pl.ds(g * NPG, NPG), :]
        h = jnp.dot(x, w1, preferred_element_type=jnp.float32)
        h = jnp.maximum(h + b1, 0.0)
        logits = jnp.dot(h, w2, preferred_element_type=jnp.float32) + b2
        m = jnp.max(logits, axis=-1, keepdims=True)
        e = jnp.exp(logits - m)
        a = e / jnp.sum(e, axis=-1, keepdims=True)
        assign_ref[pl.ds(g * NPG, NPG), :] = a
        coarse_ref[pl.ds(g * C, C), :] = lax.dot_general(
            a, x, (((0,), (0,)), ((), ())), preferred_element_type=jnp.float32)


def _stage1(nodes, w1, b1, w2, b2):
    return pl.pallas_call(
        _mlp_body,
        out_shape=[
            jax.ShapeDtypeStruct((N, C), jnp.float32),
            jax.ShapeDtypeStruct((G * C, D), jnp.float32),
        ],
    )(nodes, w1, b1, w2, b2)


def _edge_body(assign_hbm, senders_hbm, receivers_hbm, zeros_hbm, t_hbm,
               sidx_v, ridx_v, sbuf_v, rows_v, t_sh, gsem):
    cid = lax.axis_index("c")
    sid = lax.axis_index("s")
    wid = sid * NC + cid
    stripe = pl.multiple_of(sid * SRW, 8)
    pltpu.sync_copy(zeros_hbm, t_sh.at[pl.ds(stripe, SRW)])

    @pl.when(sid == NS - 1)
    def _zero_tail():
        pltpu.sync_copy(zeros_hbm.at[pl.ds(0, TAIL)],
                        t_sh.at[pl.ds(NS * SRW, TAIL)])

    pltpu.sync_copy(senders_hbm.at[pl.ds(wid * EPW, EPW)], sidx_v)
    pltpu.sync_copy(receivers_hbm.at[pl.ds(wid * EPW, EPW)], ridx_v)
    plsc.subcore_barrier()

    for b in range(NBUF):
        pltpu.make_async_copy(assign_hbm.at[ridx_v.at[pl.ds(b * CH, CH)]],
                              rows_v.at[b], gsem.at[b]).start()

    def outer(g, carry):
        for b in range(NBUF):
            c = g * NBUF + b
            pltpu.make_async_copy(assign_hbm.at[ridx_v.at[pl.ds(b * CH, CH)]],
                                  rows_v.at[b], gsem.at[b]).wait()
            for j in range(CH // 16):
                sbuf_v[pl.ds(j * 16, 16)] = sidx_v[pl.ds(c * CH + j * 16, 16)]
            pltpu.sync_copy(rows_v.at[b], t_sh.at[sbuf_v], add=True)
            cn = c + NBUF

            @pl.when(cn < NCH)
            def _prefetch():
                pltpu.make_async_copy(
                    assign_hbm.at[ridx_v.at[pl.ds(cn * CH, CH)]],
                    rows_v.at[b], gsem.at[b]).start()
        return carry

    lax.fori_loop(0, NCH // NBUF, outer, 0)
    plsc.subcore_barrier()
    pltpu.sync_copy(t_sh.at[pl.ds(stripe, SRW)],
                    t_hbm.at[cid, pl.ds(stripe, SRW)])

    @pl.when(sid == NS - 1)
    def _write_tail():
        pltpu.sync_copy(t_sh.at[pl.ds(NS * SRW, TAIL)],
                        t_hbm.at[cid, pl.ds(NS * SRW, TAIL)])


def _stage2(assignments, senders, receivers, zeros):
    run = pl.kernel(
        _edge_body,
        out_type=jax.ShapeDtypeStruct((NC, N, C), jnp.float32),
        mesh=plsc.VectorSubcoreMesh(core_axis_name="c", subcore_axis_name="s"),
        scratch_types=[
            pltpu.VMEM((EPW,), jnp.int32),
            pltpu.VMEM((EPW,), jnp.int32),
            pltpu.VMEM((CH,), jnp.int32),
            pltpu.VMEM((NBUF, CH, C), jnp.float32),
            pltpu.VMEM_SHARED((N, C), jnp.float32),
            pltpu.SemaphoreType.DMA((NBUF,)),
        ],
        compiler_params=pltpu.CompilerParams(use_tc_tiling_on_sc=False),
    )
    return run(assignments, senders, receivers, zeros)


def _adj_body(a_ref, t_ref, vals_ref, idx_ref):
    t = t_ref[0] + t_ref[1]
    a = a_ref[...]
    adjs = []
    for g in range(G):
        ag = a[g * NPG:(g + 1) * NPG]
        tg = t[g * NPG:(g + 1) * NPG]
        adjs.append(lax.dot_general(
            ag, tg, (((0,), (0,)), ((), ())),
            preferred_element_type=jnp.float32))
    work = jnp.concatenate(adjs, axis=0)
    jj = lax.broadcasted_iota(jnp.int32, (G * C, C), 1)
    vals_cols, idx_cols = [], []
    for _ in range(K):
        m = jnp.max(work, axis=-1, keepdims=True)
        eq = work == m
        idx = jnp.min(jnp.where(eq, jj, C), axis=-1, keepdims=True)
        vals_cols.append(m)
        idx_cols.append(idx)
        work = jnp.where(jj == idx, jnp.finfo(jnp.float32).min, work)
    vals_ref[...] = jnp.concatenate(vals_cols, axis=1)
    idx_ref[...] = jnp.concatenate(idx_cols, axis=1)


def _stage3(assignments, t):
    return pl.pallas_call(
        _adj_body,
        out_shape=[
            jax.ShapeDtypeStruct((G * C, K), jnp.float32),
            jax.ShapeDtypeStruct((G * C, K), jnp.int32),
        ],
    )(assignments, t)


def kernel(nodes, senders, receivers, n_node, n_edge, W1, b1, W2, b2):
    del n_node, n_edge
    assignments, coarse_nodes = _stage1(nodes, W1, b1.reshape(1, HID), W2,
                                        b2.reshape(1, C))
    zeros = jnp.zeros((SRW, C), jnp.float32)
    t = _stage2(assignments, senders, receivers, zeros)
    top_vals, top_idx = _stage3(assignments, t)
    batch_offset = jnp.arange(G, dtype=jnp.int32)[:, None] * C
    c_senders = (jnp.repeat(jnp.arange(C, dtype=jnp.int32), K)[None, :]
                 + batch_offset).reshape(-1)
    c_receivers = (top_idx.reshape(G, C * K) + batch_offset).reshape(-1)
    c_edge_weights = top_vals.reshape(-1, 1)
    return (coarse_nodes, c_senders, c_receivers, c_edge_weights,
            assignments)
